# Initial kernel scaffold; baseline (speedup 1.0000x reference)
import jax
import jax.numpy as jnp
from jax.experimental import pallas as pl


def kernel(x_target, pos_target, batch_target, x_source, pos_source, batch_source, W1, b1, W2, b2, Ws, bs):
    raise NotImplementedError("write your pallas kernel here")


# speedup vs baseline: 8.2335x; 8.2335x over previous
import functools

import jax
import jax.numpy as jnp
from jax import lax
from jax.experimental import pallas as pl
from jax.experimental.pallas import tpu as pltpu
from jax.experimental.pallas import tpu_sc as plsc

N_T, N_S, B = 16384, 4096, 4
D_T, D_S = 256, 512
C_HID, C_OUT = 512, 512
K = 3

TGT_TILE = 256
MLP_TILE = 512
BIG = 1e10


def _knn_body(pt_ref, bt_ref, ps_ref, bs_ref, idx_ref, w_ref):
    pt = pt_ref[...]
    ps = ps_ref[...]
    sq_t = jnp.sum(pt * pt, axis=1, keepdims=True)
    sq_s = jnp.sum(ps * ps, axis=0, keepdims=True)
    cross = lax.dot_general(pt, ps, (((1,), (0,)), ((), ())),
                            preferred_element_type=jnp.float32)
    dist = jnp.maximum(sq_t + sq_s - 2.0 * cross, 0.0)
    same = bt_ref[...] == bs_ref[...]
    dist = jnp.where(same, dist, jnp.float32(BIG))

    col = lax.broadcasted_iota(jnp.int32, (TGT_TILE, N_S), 1)
    ds, ids = [], []
    for _ in range(K):
        m = jnp.min(dist, axis=1, keepdims=True)
        amin = jnp.min(jnp.where(dist == m, col, jnp.int32(2**30)),
                       axis=1, keepdims=True)
        ds.append(m)
        ids.append(amin)
        dist = jnp.where(col == amin, jnp.float32(3e10), dist)
    d3 = jnp.concatenate(ds, axis=1)
    i3 = jnp.concatenate(ids, axis=1)
    w = 1.0 / jnp.maximum(d3, 1e-16)
    w = w / jnp.sum(w, axis=1, keepdims=True)
    idx_ref[...] = i3
    w_ref[...] = jnp.concatenate(
        [jnp.broadcast_to(w[:, k:k + 1], (TGT_TILE, 16)) for k in range(K)],
        axis=1)


def _knn(pos_t, bt, ps_T, bs):
    grid = N_T // TGT_TILE
    return pl.pallas_call(
        _knn_body,
        grid=(grid,),
        in_specs=[
            pl.BlockSpec((TGT_TILE, 3), lambda i: (i, 0)),
            pl.BlockSpec((TGT_TILE, 1), lambda i: (i, 0)),
            pl.BlockSpec((3, N_S), lambda i: (0, 0)),
            pl.BlockSpec((1, N_S), lambda i: (0, 0)),
        ],
        out_specs=[
            pl.BlockSpec((TGT_TILE, K), lambda i: (i, 0)),
            pl.BlockSpec((TGT_TILE, 16 * K), lambda i: (i, 0)),
        ],
        out_shape=[
            jax.ShapeDtypeStruct((N_T, K), jnp.int32),
            jax.ShapeDtypeStruct((N_T, 16 * K), jnp.float32),
        ],
    )(pos_t, bt, ps_T, bs)


_NC = 2
_NS = 16
_NW = _NC * _NS
_TPW = N_T // _NW
_TC = 16
_NCHUNK = _TPW // _TC
_ROWS = _TC * K
_NV = D_S // 16


def _sc_gather_body(xs_hbm, idx_hbm, w_hbm, out_hbm,
                    idx_v, w_v, rows_v, out_v, sem):
    wid = lax.axis_index("s") * _NC + lax.axis_index("c")
    base_t = wid * _TPW

    def chunk_body(c, carry):
        t0 = base_t + c * _TC
        pltpu.sync_copy(idx_hbm.at[pl.ds(t0 * K, _ROWS)], idx_v)
        pltpu.sync_copy(w_hbm.at[pl.ds(t0, _TC)], w_v)
        pltpu.async_copy(xs_hbm.at[idx_v], rows_v, sem).wait()

        def tgt_body(t, carry2):
            w0 = w_v[t, pl.ds(0, 16)]
            w1 = w_v[t, pl.ds(16, 16)]
            w2 = w_v[t, pl.ds(32, 16)]
            for v in range(_NV):
                sl = pl.ds(16 * v, 16)
                acc = (w0 * rows_v[3 * t, sl]
                       + w1 * rows_v[3 * t + 1, sl]
                       + w2 * rows_v[3 * t + 2, sl])
                out_v[t, sl] = acc
            return carry2

        lax.fori_loop(0, _TC, tgt_body, 0)
        pltpu.sync_copy(out_v, out_hbm.at[pl.ds(t0, _TC)])
        return carry

    lax.fori_loop(0, _NCHUNK, chunk_body, 0)


def _sc_gather(x_source, idx_flat, w_rep):
    mesh = plsc.VectorSubcoreMesh(core_axis_name="c", subcore_axis_name="s")
    kern = functools.partial(
        pl.kernel,
        mesh=mesh,
        out_type=jax.ShapeDtypeStruct((N_T, D_S), jnp.float32),
        scratch_types=[
            pltpu.VMEM((_ROWS,), jnp.int32),
            pltpu.VMEM((_TC, 16 * K), jnp.float32),
            pltpu.VMEM((_ROWS, D_S), jnp.float32),
            pltpu.VMEM((_TC, D_S), jnp.float32),
            pltpu.SemaphoreType.DMA,
        ],
    )(_sc_gather_body)
    return kern(x_source, idx_flat, w_rep)


def _mlp_body(xt_ref, it_ref, w1a_ref, w1b_ref, b1_ref,
              w2_ref, b2_ref, wsa_ref, wsb_ref, bs_ref, out_ref):
    xt = xt_ref[...]
    it = it_ref[...]
    f32 = jnp.float32
    dot = functools.partial(jnp.dot, preferred_element_type=f32)
    h = jnp.maximum(dot(xt, w1a_ref[...]) + dot(it, w1b_ref[...])
                    + b1_ref[...], 0.0)
    res = (dot(h, w2_ref[...]) + b2_ref[...]
           + dot(xt, wsa_ref[...]) + dot(it, wsb_ref[...]) + bs_ref[...])
    out_ref[...] = jnp.maximum(res, 0.0)


def _mlp(xt, it, W1a, W1b, b1, W2, b2, Wsa, Wsb, bs):
    grid = N_T // MLP_TILE
    full = lambda r, c: pl.BlockSpec((r, c), lambda i: (0, 0))
    return pl.pallas_call(
        _mlp_body,
        grid=(grid,),
        in_specs=[
            pl.BlockSpec((MLP_TILE, D_T), lambda i: (i, 0)),
            pl.BlockSpec((MLP_TILE, D_S), lambda i: (i, 0)),
            full(D_T, C_HID), full(D_S, C_HID), full(1, C_HID),
            full(C_HID, C_OUT), full(1, C_OUT),
            full(D_T, C_OUT), full(D_S, C_OUT), full(1, C_OUT),
        ],
        out_specs=pl.BlockSpec((MLP_TILE, C_OUT), lambda i: (i, 0)),
        out_shape=jax.ShapeDtypeStruct((N_T, C_OUT), jnp.float32),
    )(xt, it, W1a, W1b, b1, W2, b2, Wsa, Wsb, bs)


def kernel(x_target, pos_target, batch_target, x_source, pos_source,
           batch_source, W1, b1, W2, b2, Ws, bs):
    bt = batch_target.astype(jnp.int32).reshape(N_T, 1)
    bs_row = batch_source.astype(jnp.int32).reshape(1, N_S)
    ps_T = pos_source.T

    idx, w_rep = _knn(pos_target, bt, ps_T, bs_row)
    interp = _sc_gather(x_source, idx.reshape(-1), w_rep)

    out = _mlp(x_target, interp,
               W1[:D_T], W1[D_T:], b1.reshape(1, C_HID),
               W2, b2.reshape(1, C_OUT),
               Ws[:D_T], Ws[D_T:], bs.reshape(1, C_OUT))
    return out


# speedup vs baseline: 10.6989x; 1.2994x over previous
import functools

import jax
import jax.numpy as jnp
import numpy as np
from jax import lax
from jax.experimental import pallas as pl
from jax.experimental.pallas import tpu as pltpu
from jax.experimental.pallas import tpu_sc as plsc

N_T, N_S, B = 16384, 4096, 4
D_T, D_S = 256, 512
C_HID, C_OUT = 512, 512
K = 3

TGT_TILE = 256
MLP_TILE = 512
BIG = 1e10


SRC_BLK = 512
N_SRC_BLK = N_S // SRC_BLK


def _knn_body(ptT_ref, bt_ref, ps_ref, bs_ref, idx_ref, w_ref):
    ptT = ptT_ref[...]
    bt = bt_ref[...]
    bs = bs_ref[...]
    sq_t = jnp.sum(ptT * ptT, axis=0, keepdims=True)

    f32 = jnp.float32
    lo_t = jnp.zeros((1, TGT_TILE), f32)
    hi_t = jnp.zeros((1, TGT_TILE), f32)
    cum = jnp.int32(0)
    for b in range(B):
        sel_b = bt == b
        lo_t = jnp.where(sel_b, cum.astype(f32), lo_t)
        cum = cum + jnp.sum((bs == b).astype(jnp.int32))
        hi_t = jnp.where(sel_b, cum.astype(f32), hi_t)
    b_first = jnp.min(bt)
    b_last = jnp.max(bt)
    lo_s = jnp.sum((bs < b_first).astype(jnp.int32))
    hi_s = jnp.sum((bs <= b_last).astype(jnp.int32))
    lo_blk = lo_s // SRC_BLK
    hi_blk = (hi_s + SRC_BLK - 1) // SRC_BLK

    INF = float(np.inf)
    rd = [jnp.full((1, TGT_TILE), BIG, f32) for _ in range(K)]
    ri = [jnp.full((1, TGT_TILE), float(k), f32) for k in range(K)]
    row_iota = lax.broadcasted_iota(
        jnp.int32, (SRC_BLK, TGT_TILE), 0).astype(f32)

    def block_body(j, carry):
        rd1, ri1, rd2, ri2, rd3, ri3 = carry
        psj = ps_ref[pl.ds(j * SRC_BLK, SRC_BLK), :]
        cross = lax.dot_general(psj, ptT, (((1,), (0,)), ((), ())),
                                preferred_element_type=f32)
        sq_s = jnp.sum(psj * psj, axis=1, keepdims=True)
        d = jnp.maximum(sq_s + sq_t - 2.0 * cross, 0.0)
        gidx = row_iota + (j * SRC_BLK).astype(f32)
        in_range = (gidx >= lo_t) & (gidx < hi_t)
        d = jnp.where(in_range, d, INF)

        for _ in range(K):
            m = jnp.min(d, axis=0, keepdims=True)
            am = jnp.min(jnp.where(d == m, gidx, f32(float(N_S))),
                         axis=0, keepdims=True)
            d = jnp.where(gidx == am, INF, d)
            l1 = (m < rd1) | ((m == rd1) & (am < ri1))
            l2 = (m < rd2) | ((m == rd2) & (am < ri2))
            l3 = (m < rd3) | ((m == rd3) & (am < ri3))
            rd3 = jnp.where(l2, rd2, jnp.where(l3, m, rd3))
            ri3 = jnp.where(l2, ri2, jnp.where(l3, am, ri3))
            rd2 = jnp.where(l1, rd1, jnp.where(l2, m, rd2))
            ri2 = jnp.where(l1, ri1, jnp.where(l2, am, ri2))
            rd1 = jnp.where(l1, m, rd1)
            ri1 = jnp.where(l1, am, ri1)
        return (rd1, ri1, rd2, ri2, rd3, ri3)

    rd1, ri1, rd2, ri2, rd3, ri3 = lax.fori_loop(
        lo_blk, hi_blk, block_body,
        (rd[0], ri[0], rd[1], ri[1], rd[2], ri[2]))

    d3 = jnp.concatenate([rd1, rd2, rd3], axis=0)
    i3 = jnp.concatenate([ri1, ri2, ri3], axis=0)
    w = 1.0 / jnp.maximum(d3, 1e-16)
    w = w / jnp.sum(w, axis=0, keepdims=True)
    idx_ref[...] = jnp.transpose(i3).astype(jnp.int32)
    wT = jnp.transpose(w)
    w_ref[...] = jnp.concatenate(
        [jnp.broadcast_to(wT[:, k:k + 1], (TGT_TILE, 16)) for k in range(K)],
        axis=1)


def _knn(pos_tT, bt, ps, bs):
    grid = N_T // TGT_TILE
    return pl.pallas_call(
        _knn_body,
        grid=(grid,),
        in_specs=[
            pl.BlockSpec((3, TGT_TILE), lambda i: (0, i)),
            pl.BlockSpec((1, TGT_TILE), lambda i: (0, i)),
            pl.BlockSpec((N_S, 3), lambda i: (0, 0)),
            pl.BlockSpec((1, N_S), lambda i: (0, 0)),
        ],
        out_specs=[
            pl.BlockSpec((TGT_TILE, K), lambda i: (i, 0)),
            pl.BlockSpec((TGT_TILE, 16 * K), lambda i: (i, 0)),
        ],
        out_shape=[
            jax.ShapeDtypeStruct((N_T, K), jnp.int32),
            jax.ShapeDtypeStruct((N_T, 16 * K), jnp.float32),
        ],
    )(pos_tT, bt, ps, bs)


_NC = 2
_NS = 16
_NW = _NC * _NS
_TPW = N_T // _NW
_TC = 16
_NCHUNK = _TPW // _TC
_ROWS = _TC * K
_NV = D_S // 16


def _sc_gather_body(xs_hbm, idx_hbm, w_hbm, out_hbm,
                    idx_v, w_v, rows_v, out_v, sem):
    wid = lax.axis_index("s") * _NC + lax.axis_index("c")
    base_t = wid * _TPW

    def chunk_body(c, carry):
        t0 = base_t + c * _TC
        pltpu.sync_copy(idx_hbm.at[pl.ds(t0 * K, _ROWS)], idx_v)
        pltpu.sync_copy(w_hbm.at[pl.ds(t0, _TC)], w_v)
        pltpu.async_copy(xs_hbm.at[idx_v], rows_v, sem).wait()

        def tgt_body(t, carry2):
            w0 = w_v[t, pl.ds(0, 16)]
            w1 = w_v[t, pl.ds(16, 16)]
            w2 = w_v[t, pl.ds(32, 16)]
            for v in range(_NV):
                sl = pl.ds(16 * v, 16)
                acc = (w0 * rows_v[3 * t, sl]
                       + w1 * rows_v[3 * t + 1, sl]
                       + w2 * rows_v[3 * t + 2, sl])
                out_v[t, sl] = acc
            return carry2

        lax.fori_loop(0, _TC, tgt_body, 0)
        pltpu.sync_copy(out_v, out_hbm.at[pl.ds(t0, _TC)])
        return carry

    lax.fori_loop(0, _NCHUNK, chunk_body, 0)


def _sc_gather(x_source, idx_flat, w_rep):
    mesh = plsc.VectorSubcoreMesh(core_axis_name="c", subcore_axis_name="s")
    kern = functools.partial(
        pl.kernel,
        mesh=mesh,
        out_type=jax.ShapeDtypeStruct((N_T, D_S), jnp.float32),
        scratch_types=[
            pltpu.VMEM((_ROWS,), jnp.int32),
            pltpu.VMEM((_TC, 16 * K), jnp.float32),
            pltpu.VMEM((_ROWS, D_S), jnp.float32),
            pltpu.VMEM((_TC, D_S), jnp.float32),
            pltpu.SemaphoreType.DMA,
        ],
    )(_sc_gather_body)
    return kern(x_source, idx_flat, w_rep)


def _mlp_body(xt_ref, it_ref, w1a_ref, w1b_ref, b1_ref,
              w2_ref, b2_ref, wsa_ref, wsb_ref, bs_ref, out_ref):
    xt = xt_ref[...]
    it = it_ref[...]
    f32 = jnp.float32
    dot = functools.partial(jnp.dot, preferred_element_type=f32)
    h = jnp.maximum(dot(xt, w1a_ref[...]) + dot(it, w1b_ref[...])
                    + b1_ref[...], 0.0)
    res = (dot(h, w2_ref[...]) + b2_ref[...]
           + dot(xt, wsa_ref[...]) + dot(it, wsb_ref[...]) + bs_ref[...])
    out_ref[...] = jnp.maximum(res, 0.0)


def _mlp(xt, it, W1a, W1b, b1, W2, b2, Wsa, Wsb, bs):
    grid = N_T // MLP_TILE
    full = lambda r, c: pl.BlockSpec((r, c), lambda i: (0, 0))
    return pl.pallas_call(
        _mlp_body,
        grid=(grid,),
        in_specs=[
            pl.BlockSpec((MLP_TILE, D_T), lambda i: (i, 0)),
            pl.BlockSpec((MLP_TILE, D_S), lambda i: (i, 0)),
            full(D_T, C_HID), full(D_S, C_HID), full(1, C_HID),
            full(C_HID, C_OUT), full(1, C_OUT),
            full(D_T, C_OUT), full(D_S, C_OUT), full(1, C_OUT),
        ],
        out_specs=pl.BlockSpec((MLP_TILE, C_OUT), lambda i: (i, 0)),
        out_shape=jax.ShapeDtypeStruct((N_T, C_OUT), jnp.float32),
    )(xt, it, W1a, W1b, b1, W2, b2, Wsa, Wsb, bs)


def kernel(x_target, pos_target, batch_target, x_source, pos_source,
           batch_source, W1, b1, W2, b2, Ws, bs):
    bt_row = batch_target.astype(jnp.int32).reshape(1, N_T)
    bs_row = batch_source.astype(jnp.int32).reshape(1, N_S)
    pt_T = pos_target.T

    idx, w_rep = _knn(pt_T, bt_row, pos_source, bs_row)
    interp = _sc_gather(x_source, idx.reshape(-1), w_rep)

    out = _mlp(x_target, interp,
               W1[:D_T], W1[D_T:], b1.reshape(1, C_HID),
               W2, b2.reshape(1, C_OUT),
               Ws[:D_T], Ws[D_T:], bs.reshape(1, C_OUT))
    return out


# speedup vs baseline: 12.4542x; 1.1641x over previous
import functools

import jax
import jax.numpy as jnp
import numpy as np
from jax import lax
from jax.experimental import pallas as pl
from jax.experimental.pallas import tpu as pltpu
from jax.experimental.pallas import tpu_sc as plsc

N_T, N_S, B = 16384, 4096, 4
D_T, D_S = 256, 512
C_HID, C_OUT = 512, 512
K = 3

TGT_TILE = 256
MLP_TILE = 512
BIG = 1e10


SRC_BLK = 512
N_SRC_BLK = N_S // SRC_BLK


def _knn_body(ptT_ref, bt_ref, ps_ref, bs_ref, idx_ref, w_ref):
    ptT = ptT_ref[...]
    bt = bt_ref[...]
    bs = bs_ref[...]
    sq_t = jnp.sum(ptT * ptT, axis=0, keepdims=True)

    f32 = jnp.float32
    lo_t = jnp.zeros((1, TGT_TILE), f32)
    hi_t = jnp.zeros((1, TGT_TILE), f32)
    cum = jnp.int32(0)
    for b in range(B):
        sel_b = bt == b
        lo_t = jnp.where(sel_b, cum.astype(f32), lo_t)
        cum = cum + jnp.sum((bs == b).astype(jnp.int32))
        hi_t = jnp.where(sel_b, cum.astype(f32), hi_t)
    b_first = jnp.min(bt)
    b_last = jnp.max(bt)
    lo_s = jnp.sum((bs < b_first).astype(jnp.int32))
    hi_s = jnp.sum((bs <= b_last).astype(jnp.int32))
    lo_blk = lo_s // SRC_BLK
    hi_blk = (hi_s + SRC_BLK - 1) // SRC_BLK

    INF = float(np.inf)
    rd = [jnp.full((1, TGT_TILE), BIG, f32) for _ in range(K)]
    ri = [jnp.full((1, TGT_TILE), float(k), f32) for k in range(K)]
    row_iota = lax.broadcasted_iota(
        jnp.int32, (SRC_BLK, TGT_TILE), 0).astype(f32)

    def block_body(j, carry):
        rd1, ri1, rd2, ri2, rd3, ri3 = carry
        psj = ps_ref[pl.ds(j * SRC_BLK, SRC_BLK), :]
        cross = lax.dot_general(psj, ptT, (((1,), (0,)), ((), ())),
                                preferred_element_type=f32)
        sq_s = jnp.sum(psj * psj, axis=1, keepdims=True)
        d = jnp.maximum(sq_s + sq_t - 2.0 * cross, 0.0)
        gidx = row_iota + (j * SRC_BLK).astype(f32)
        in_range = (gidx >= lo_t) & (gidx < hi_t)
        d = jnp.where(in_range, d, INF)

        for _ in range(K):
            m = jnp.min(d, axis=0, keepdims=True)
            am = jnp.min(jnp.where(d == m, gidx, f32(float(N_S))),
                         axis=0, keepdims=True)
            d = jnp.where(gidx == am, INF, d)
            l1 = (m < rd1) | ((m == rd1) & (am < ri1))
            l2 = (m < rd2) | ((m == rd2) & (am < ri2))
            l3 = (m < rd3) | ((m == rd3) & (am < ri3))
            rd3 = jnp.where(l2, rd2, jnp.where(l3, m, rd3))
            ri3 = jnp.where(l2, ri2, jnp.where(l3, am, ri3))
            rd2 = jnp.where(l1, rd1, jnp.where(l2, m, rd2))
            ri2 = jnp.where(l1, ri1, jnp.where(l2, am, ri2))
            rd1 = jnp.where(l1, m, rd1)
            ri1 = jnp.where(l1, am, ri1)
        return (rd1, ri1, rd2, ri2, rd3, ri3)

    rd1, ri1, rd2, ri2, rd3, ri3 = lax.fori_loop(
        lo_blk, hi_blk, block_body,
        (rd[0], ri[0], rd[1], ri[1], rd[2], ri[2]))

    d3 = jnp.concatenate([rd1, rd2, rd3], axis=0)
    i3 = jnp.concatenate([ri1, ri2, ri3], axis=0)
    w = 1.0 / jnp.maximum(d3, 1e-16)
    w = w / jnp.sum(w, axis=0, keepdims=True)
    idx_ref[...] = jnp.transpose(i3).astype(jnp.int32)
    wT = jnp.transpose(w)
    w_ref[...] = jnp.concatenate(
        [jnp.broadcast_to(wT[:, k:k + 1], (TGT_TILE, 16)) for k in range(K)],
        axis=1)


def _knn(pos_tT, bt, ps, bs):
    grid = N_T // TGT_TILE
    return pl.pallas_call(
        _knn_body,
        grid=(grid,),
        in_specs=[
            pl.BlockSpec((3, TGT_TILE), lambda i: (0, i)),
            pl.BlockSpec((1, TGT_TILE), lambda i: (0, i)),
            pl.BlockSpec((N_S, 3), lambda i: (0, 0)),
            pl.BlockSpec((1, N_S), lambda i: (0, 0)),
        ],
        out_specs=[
            pl.BlockSpec((TGT_TILE, K), lambda i: (i, 0)),
            pl.BlockSpec((TGT_TILE, 16 * K), lambda i: (i, 0)),
        ],
        out_shape=[
            jax.ShapeDtypeStruct((N_T, K), jnp.int32),
            jax.ShapeDtypeStruct((N_T, 16 * K), jnp.float32),
        ],
    )(pos_tT, bt, ps, bs)


_NC = 2
_NS = 16
_NW = _NC * _NS
_TPW = N_T // _NW
_TC = 16
_NCHUNK = _TPW // _TC
_NPAIR = _NCHUNK // 2
_ROWS = _TC * K
_NV = D_S // 16


def _sc_gather_body(xs_hbm, idx_hbm, w_hbm, out_hbm,
                    idx_v0, w_v0, rows_v0, out_v0,
                    idx_v1, w_v1, rows_v1, out_v1,
                    si0, sw0, sg0, so0, si1, sw1, sg1, so1):
    wid = lax.axis_index("s") * _NC + lax.axis_index("c")
    base_t = wid * _TPW
    bufs = ((idx_v0, w_v0, rows_v0, out_v0, si0, sw0, sg0, so0),
            (idx_v1, w_v1, rows_v1, out_v1, si1, sw1, sg1, so1))

    def meta_copies(c, b):
        idx_v, w_v = bufs[b][0], bufs[b][1]
        t0 = base_t + c * _TC
        ci = pltpu.make_async_copy(idx_hbm.at[pl.ds(t0 * K, _ROWS)], idx_v,
                                   bufs[b][4])
        cw = pltpu.make_async_copy(w_hbm.at[pl.ds(t0, _TC)], w_v, bufs[b][5])
        return ci, cw

    def gather_copy(b):
        return pltpu.make_async_copy(xs_hbm.at[bufs[b][0]], bufs[b][2],
                                     bufs[b][6])

    def out_copy(c, b):
        return pltpu.make_async_copy(bufs[b][3],
                                     out_hbm.at[pl.ds(base_t + c * _TC, _TC)],
                                     bufs[b][7])

    def compute(c, b, p):
        w_v, rows_v, out_v = bufs[b][1], bufs[b][2], bufs[b][3]

        @pl.when(p > 0)
        def _():
            out_copy(c, b).wait()

        def tgt_body(t, carry):
            w0 = w_v[t, pl.ds(0, 16)]
            w1 = w_v[t, pl.ds(16, 16)]
            w2 = w_v[t, pl.ds(32, 16)]
            for v in range(_NV):
                sl = pl.ds(16 * v, 16)
                out_v[t, sl] = (w0 * rows_v[3 * t, sl]
                                + w1 * rows_v[3 * t + 1, sl]
                                + w2 * rows_v[3 * t + 2, sl])
            return carry

        lax.fori_loop(0, _TC, tgt_body, 0)
        out_copy(c, b).start()

    ci, cw = meta_copies(0, 0)
    ci.start(); cw.start(); ci.wait(); cw.wait()
    gather_copy(0).start()
    ci, cw = meta_copies(1, 1)
    ci.start(); cw.start()

    def pair_body(p, carry):
        c0 = 2 * p
        ci, cw = meta_copies(c0 + 1, 1)
        ci.wait(); cw.wait()
        gather_copy(1).start()
        gather_copy(0).wait()
        compute(c0, 0, p)

        @pl.when(p < _NPAIR - 1)
        def _():
            ci, cw = meta_copies(c0 + 2, 0)
            ci.start(); cw.start()

        gather_copy(1).wait()
        compute(c0 + 1, 1, p)

        @pl.when(p < _NPAIR - 1)
        def _():
            ci, cw = meta_copies(c0 + 3, 1)
            ci.start(); cw.start()

        @pl.when(p < _NPAIR - 1)
        def _():
            ci, cw = meta_copies(c0 + 2, 0)
            ci.wait(); cw.wait()
            gather_copy(0).start()
        return carry

    lax.fori_loop(0, _NPAIR, pair_body, 0)
    out_copy(_NCHUNK - 2, 0).wait()
    out_copy(_NCHUNK - 1, 1).wait()


def _sc_gather(x_source, idx_flat, w_rep):
    mesh = plsc.VectorSubcoreMesh(core_axis_name="c", subcore_axis_name="s")
    buf = [
        pltpu.VMEM((_ROWS,), jnp.int32),
        pltpu.VMEM((_TC, 16 * K), jnp.float32),
        pltpu.VMEM((_ROWS, D_S), jnp.float32),
        pltpu.VMEM((_TC, D_S), jnp.float32),
    ]
    sems = [pltpu.SemaphoreType.DMA] * 4
    kern = functools.partial(
        pl.kernel,
        mesh=mesh,
        out_type=jax.ShapeDtypeStruct((N_T, D_S), jnp.float32),
        scratch_types=buf + buf + sems + sems,
    )(_sc_gather_body)
    return kern(x_source, idx_flat, w_rep)


def _mlp_body(xt_ref, it_ref, w1a_ref, w1b_ref, b1_ref,
              w2_ref, b2_ref, wsa_ref, wsb_ref, bs_ref, out_ref):
    xt = xt_ref[...]
    it = it_ref[...]
    f32 = jnp.float32
    dot = functools.partial(jnp.dot, preferred_element_type=f32)
    h = jnp.maximum(dot(xt, w1a_ref[...]) + dot(it, w1b_ref[...])
                    + b1_ref[...], 0.0)
    res = (dot(h, w2_ref[...]) + b2_ref[...]
           + dot(xt, wsa_ref[...]) + dot(it, wsb_ref[...]) + bs_ref[...])
    out_ref[...] = jnp.maximum(res, 0.0)


def _mlp(xt, it, W1a, W1b, b1, W2, b2, Wsa, Wsb, bs):
    grid = N_T // MLP_TILE
    full = lambda r, c: pl.BlockSpec((r, c), lambda i: (0, 0))
    return pl.pallas_call(
        _mlp_body,
        grid=(grid,),
        in_specs=[
            pl.BlockSpec((MLP_TILE, D_T), lambda i: (i, 0)),
            pl.BlockSpec((MLP_TILE, D_S), lambda i: (i, 0)),
            full(D_T, C_HID), full(D_S, C_HID), full(1, C_HID),
            full(C_HID, C_OUT), full(1, C_OUT),
            full(D_T, C_OUT), full(D_S, C_OUT), full(1, C_OUT),
        ],
        out_specs=pl.BlockSpec((MLP_TILE, C_OUT), lambda i: (i, 0)),
        out_shape=jax.ShapeDtypeStruct((N_T, C_OUT), jnp.float32),
    )(xt, it, W1a, W1b, b1, W2, b2, Wsa, Wsb, bs)


def kernel(x_target, pos_target, batch_target, x_source, pos_source,
           batch_source, W1, b1, W2, b2, Ws, bs):
    bt_row = batch_target.astype(jnp.int32).reshape(1, N_T)
    bs_row = batch_source.astype(jnp.int32).reshape(1, N_S)
    pt_T = pos_target.T

    idx, w_rep = _knn(pt_T, bt_row, pos_source, bs_row)
    interp = _sc_gather(x_source, idx.reshape(-1), w_rep)

    out = _mlp(x_target, interp,
               W1[:D_T], W1[D_T:], b1.reshape(1, C_HID),
               W2, b2.reshape(1, C_OUT),
               Ws[:D_T], Ws[D_T:], bs.reshape(1, C_OUT))
    return out


# speedup vs baseline: 15.0608x; 1.2093x over previous
import functools

import jax
import jax.numpy as jnp
import numpy as np
from jax import lax
from jax.experimental import pallas as pl
from jax.experimental.pallas import tpu as pltpu
from jax.experimental.pallas import tpu_sc as plsc

N_T, N_S, B = 16384, 4096, 4
D_T, D_S = 256, 512
C_HID, C_OUT = 512, 512
K = 3

TGT_TILE = 256
MLP_TILE = 512
BIG = 1e10


SRC_BLK = 512
N_SRC_BLK = N_S // SRC_BLK


def _knn_body(ptT_ref, bt_ref, ps_ref, bs_ref, idx_ref, w_ref):
    ptT = ptT_ref[...]
    bt = bt_ref[...]
    bs = bs_ref[...]
    sq_t = jnp.sum(ptT * ptT, axis=0, keepdims=True)

    f32 = jnp.float32
    lo_t = jnp.zeros((1, TGT_TILE), f32)
    hi_t = jnp.zeros((1, TGT_TILE), f32)
    cum = jnp.int32(0)
    for b in range(B):
        sel_b = bt == b
        lo_t = jnp.where(sel_b, cum.astype(f32), lo_t)
        cum = cum + jnp.sum((bs == b).astype(jnp.int32))
        hi_t = jnp.where(sel_b, cum.astype(f32), hi_t)
    b_first = jnp.min(bt)
    b_last = jnp.max(bt)
    lo_s = jnp.sum((bs < b_first).astype(jnp.int32))
    hi_s = jnp.sum((bs <= b_last).astype(jnp.int32))
    lo_blk = lo_s // SRC_BLK
    hi_blk = (hi_s + SRC_BLK - 1) // SRC_BLK

    INF = float(np.inf)
    rd = [jnp.full((1, TGT_TILE), BIG, f32) for _ in range(K)]
    ri = [jnp.full((1, TGT_TILE), float(k), f32) for k in range(K)]
    row_iota = lax.broadcasted_iota(
        jnp.int32, (SRC_BLK, TGT_TILE), 0).astype(f32)

    def block_body(j, carry):
        rd1, ri1, rd2, ri2, rd3, ri3 = carry
        psj = ps_ref[pl.ds(j * SRC_BLK, SRC_BLK), :]
        cross = lax.dot_general(psj, ptT, (((1,), (0,)), ((), ())),
                                preferred_element_type=f32)
        sq_s = jnp.sum(psj * psj, axis=1, keepdims=True)
        d = jnp.maximum(sq_s + sq_t - 2.0 * cross, 0.0)
        gidx = row_iota + (j * SRC_BLK).astype(f32)
        in_range = (gidx >= lo_t) & (gidx < hi_t)
        d = jnp.where(in_range, d, INF)

        for _ in range(K):
            m = jnp.min(d, axis=0, keepdims=True)
            am = jnp.min(jnp.where(d == m, gidx, f32(float(N_S))),
                         axis=0, keepdims=True)
            d = jnp.where(gidx == am, INF, d)
            l1 = (m < rd1) | ((m == rd1) & (am < ri1))
            l2 = (m < rd2) | ((m == rd2) & (am < ri2))
            l3 = (m < rd3) | ((m == rd3) & (am < ri3))
            rd3 = jnp.where(l2, rd2, jnp.where(l3, m, rd3))
            ri3 = jnp.where(l2, ri2, jnp.where(l3, am, ri3))
            rd2 = jnp.where(l1, rd1, jnp.where(l2, m, rd2))
            ri2 = jnp.where(l1, ri1, jnp.where(l2, am, ri2))
            rd1 = jnp.where(l1, m, rd1)
            ri1 = jnp.where(l1, am, ri1)
        return (rd1, ri1, rd2, ri2, rd3, ri3)

    rd1, ri1, rd2, ri2, rd3, ri3 = lax.fori_loop(
        lo_blk, hi_blk, block_body,
        (rd[0], ri[0], rd[1], ri[1], rd[2], ri[2]))

    d3 = jnp.concatenate([rd1, rd2, rd3], axis=0)
    i3 = jnp.concatenate([ri1, ri2, ri3], axis=0)
    w = 1.0 / jnp.maximum(d3, 1e-16)
    w = w / jnp.sum(w, axis=0, keepdims=True)
    idx_ref[...] = jnp.transpose(i3).astype(jnp.int32)
    wT = jnp.transpose(w)
    w_ref[...] = jnp.concatenate(
        [jnp.broadcast_to(wT[:, k:k + 1], (TGT_TILE, 16)) for k in range(K)],
        axis=1)


def _knn(pos_tT, bt, ps, bs):
    n = pos_tT.shape[1]
    grid = n // TGT_TILE
    return pl.pallas_call(
        _knn_body,
        grid=(grid,),
        in_specs=[
            pl.BlockSpec((3, TGT_TILE), lambda i: (0, i)),
            pl.BlockSpec((1, TGT_TILE), lambda i: (0, i)),
            pl.BlockSpec((N_S, 3), lambda i: (0, 0)),
            pl.BlockSpec((1, N_S), lambda i: (0, 0)),
        ],
        out_specs=[
            pl.BlockSpec((TGT_TILE, K), lambda i: (i, 0)),
            pl.BlockSpec((TGT_TILE, 16 * K), lambda i: (i, 0)),
        ],
        out_shape=[
            jax.ShapeDtypeStruct((n, K), jnp.int32),
            jax.ShapeDtypeStruct((n, 16 * K), jnp.float32),
        ],
    )(pos_tT, bt, ps, bs)


_NC = 2
_NS = 16
_NW = _NC * _NS
_TC = 16
_ROWS = _TC * K
_NV = D_S // 16


def _make_sc_body(tpw, nchunk, npair):
  def _sc_gather_body(xs_hbm, idx_hbm, w_hbm, out_hbm,
                      idx_v0, w_v0, rows_v0, out_v0,
                      idx_v1, w_v1, rows_v1, out_v1,
                      si0, sw0, sg0, so0, si1, sw1, sg1, so1):
    wid = lax.axis_index("s") * _NC + lax.axis_index("c")
    base_t = wid * tpw
    bufs = ((idx_v0, w_v0, rows_v0, out_v0, si0, sw0, sg0, so0),
            (idx_v1, w_v1, rows_v1, out_v1, si1, sw1, sg1, so1))

    def meta_copies(c, b):
        idx_v, w_v = bufs[b][0], bufs[b][1]
        t0 = base_t + c * _TC
        ci = pltpu.make_async_copy(idx_hbm.at[pl.ds(t0 * K, _ROWS)], idx_v,
                                   bufs[b][4])
        cw = pltpu.make_async_copy(w_hbm.at[pl.ds(t0, _TC)], w_v, bufs[b][5])
        return ci, cw

    def gather_copy(b):
        return pltpu.make_async_copy(xs_hbm.at[bufs[b][0]], bufs[b][2],
                                     bufs[b][6])

    def out_copy(c, b):
        return pltpu.make_async_copy(bufs[b][3],
                                     out_hbm.at[pl.ds(base_t + c * _TC, _TC)],
                                     bufs[b][7])

    def compute(c, b, p):
        w_v, rows_v, out_v = bufs[b][1], bufs[b][2], bufs[b][3]

        @pl.when(p > 0)
        def _():
            out_copy(c, b).wait()

        def tgt_body(t, carry):
            w0 = w_v[t, pl.ds(0, 16)]
            w1 = w_v[t, pl.ds(16, 16)]
            w2 = w_v[t, pl.ds(32, 16)]
            for v in range(_NV):
                sl = pl.ds(16 * v, 16)
                out_v[t, sl] = (w0 * rows_v[3 * t, sl]
                                + w1 * rows_v[3 * t + 1, sl]
                                + w2 * rows_v[3 * t + 2, sl])
            return carry

        lax.fori_loop(0, _TC, tgt_body, 0)
        out_copy(c, b).start()

    ci, cw = meta_copies(0, 0)
    ci.start(); cw.start(); ci.wait(); cw.wait()
    gather_copy(0).start()
    ci, cw = meta_copies(1, 1)
    ci.start(); cw.start()

    def pair_body(p, carry):
        c0 = 2 * p
        ci, cw = meta_copies(c0 + 1, 1)
        ci.wait(); cw.wait()
        gather_copy(1).start()
        gather_copy(0).wait()
        compute(c0, 0, p)

        @pl.when(p < npair - 1)
        def _():
            ci, cw = meta_copies(c0 + 2, 0)
            ci.start(); cw.start()

        gather_copy(1).wait()
        compute(c0 + 1, 1, p)

        @pl.when(p < npair - 1)
        def _():
            ci, cw = meta_copies(c0 + 3, 1)
            ci.start(); cw.start()

        @pl.when(p < npair - 1)
        def _():
            ci, cw = meta_copies(c0 + 2, 0)
            ci.wait(); cw.wait()
            gather_copy(0).start()
        return carry

    lax.fori_loop(0, npair, pair_body, 0)
    out_copy(nchunk - 2, 0).wait()
    out_copy(nchunk - 1, 1).wait()

  return _sc_gather_body


def _sc_gather(x_source, idx_flat, w_rep):
    n = w_rep.shape[0]
    tpw = n // _NW
    nchunk = tpw // _TC
    mesh = plsc.VectorSubcoreMesh(core_axis_name="c", subcore_axis_name="s")
    buf = [
        pltpu.VMEM((_ROWS,), jnp.int32),
        pltpu.VMEM((_TC, 16 * K), jnp.float32),
        pltpu.VMEM((_ROWS, D_S), jnp.float32),
        pltpu.VMEM((_TC, D_S), jnp.float32),
    ]
    sems = [pltpu.SemaphoreType.DMA] * 4
    kern = functools.partial(
        pl.kernel,
        mesh=mesh,
        out_type=jax.ShapeDtypeStruct((n, D_S), jnp.float32),
        scratch_types=buf + buf + sems + sems,
    )(_make_sc_body(tpw, nchunk, nchunk // 2))
    return kern(x_source, idx_flat, w_rep)


def _mlp_body(xt_ref, it_ref, w1a_ref, w1b_ref, b1_ref,
              w2_ref, b2_ref, wsa_ref, wsb_ref, bs_ref, out_ref):
    xt = xt_ref[...]
    it = it_ref[...]
    f32 = jnp.float32
    dot = functools.partial(jnp.dot, preferred_element_type=f32)
    h = jnp.maximum(dot(xt, w1a_ref[...]) + dot(it, w1b_ref[...])
                    + b1_ref[...], 0.0)
    res = (dot(h, w2_ref[...]) + b2_ref[...]
           + dot(xt, wsa_ref[...]) + dot(it, wsb_ref[...]) + bs_ref[...])
    out_ref[...] = jnp.maximum(res, 0.0)


def _mlp(xt, it, W1a, W1b, b1, W2, b2, Wsa, Wsb, bs):
    n = xt.shape[0]
    grid = n // MLP_TILE
    full = lambda r, c: pl.BlockSpec((r, c), lambda i: (0, 0))
    return pl.pallas_call(
        _mlp_body,
        grid=(grid,),
        in_specs=[
            pl.BlockSpec((MLP_TILE, D_T), lambda i: (i, 0)),
            pl.BlockSpec((MLP_TILE, D_S), lambda i: (i, 0)),
            full(D_T, C_HID), full(D_S, C_HID), full(1, C_HID),
            full(C_HID, C_OUT), full(1, C_OUT),
            full(D_T, C_OUT), full(D_S, C_OUT), full(1, C_OUT),
        ],
        out_specs=pl.BlockSpec((MLP_TILE, C_OUT), lambda i: (i, 0)),
        out_shape=jax.ShapeDtypeStruct((n, C_OUT), jnp.float32),
    )(xt, it, W1a, W1b, b1, W2, b2, Wsa, Wsb, bs)


def kernel(x_target, pos_target, batch_target, x_source, pos_source,
           batch_source, W1, b1, W2, b2, Ws, bs):
    bt_row = batch_target.astype(jnp.int32).reshape(1, N_T)
    bs_row = batch_source.astype(jnp.int32).reshape(1, N_S)
    pt_T = pos_target.T

    HALF = N_T // 2
    outs = []
    for h in range(2):
        s = slice(h * HALF, (h + 1) * HALF)
        idx, w_rep = _knn(pt_T[:, s], bt_row[:, s], pos_source, bs_row)
        interp = _sc_gather(x_source, idx.reshape(-1), w_rep)
        outs.append(_mlp(x_target[s], interp,
                         W1[:D_T], W1[D_T:], b1.reshape(1, C_HID),
                         W2, b2.reshape(1, C_OUT),
                         Ws[:D_T], Ws[D_T:], bs.reshape(1, C_OUT)))
    return jnp.concatenate(outs, axis=0)


# speedup vs baseline: 15.5151x; 1.0302x over previous
import functools

import jax
import jax.numpy as jnp
import numpy as np
from jax import lax
from jax.experimental import pallas as pl
from jax.experimental.pallas import tpu as pltpu
from jax.experimental.pallas import tpu_sc as plsc

N_T, N_S, B = 16384, 4096, 4
D_T, D_S = 256, 512
C_HID, C_OUT = 512, 512
K = 3

TGT_TILE = 256
MLP_TILE = 512
BIG = 1e10


SRC_BLK = 512
N_SRC_BLK = N_S // SRC_BLK


def _knn_body(ptT_ref, bt_ref, ps_ref, bs_ref, idx_ref, w_ref):
    ptT = ptT_ref[...]
    bt = bt_ref[...]
    bs = bs_ref[...]
    sq_t = jnp.sum(ptT * ptT, axis=0, keepdims=True)

    f32 = jnp.float32
    lo_t = jnp.zeros((1, TGT_TILE), f32)
    hi_t = jnp.zeros((1, TGT_TILE), f32)
    cum = jnp.int32(0)
    for b in range(B):
        sel_b = bt == b
        lo_t = jnp.where(sel_b, cum.astype(f32), lo_t)
        cum = cum + jnp.sum((bs == b).astype(jnp.int32))
        hi_t = jnp.where(sel_b, cum.astype(f32), hi_t)
    b_first = jnp.min(bt)
    b_last = jnp.max(bt)
    lo_s = jnp.sum((bs < b_first).astype(jnp.int32))
    hi_s = jnp.sum((bs <= b_last).astype(jnp.int32))
    lo_blk = lo_s // SRC_BLK
    hi_blk = (hi_s + SRC_BLK - 1) // SRC_BLK

    INF = float(np.inf)
    rd = [jnp.full((1, TGT_TILE), BIG, f32) for _ in range(K)]
    ri = [jnp.full((1, TGT_TILE), float(k), f32) for k in range(K)]
    row_iota = lax.broadcasted_iota(
        jnp.int32, (SRC_BLK, TGT_TILE), 0).astype(f32)

    def block_body(j, carry):
        rd1, ri1, rd2, ri2, rd3, ri3 = carry
        psj = ps_ref[pl.ds(j * SRC_BLK, SRC_BLK), :]
        cross = lax.dot_general(psj, ptT, (((1,), (0,)), ((), ())),
                                preferred_element_type=f32)
        sq_s = jnp.sum(psj * psj, axis=1, keepdims=True)
        d = jnp.maximum(sq_s + sq_t - 2.0 * cross, 0.0)
        gidx = row_iota + (j * SRC_BLK).astype(f32)
        in_range = (gidx >= lo_t) & (gidx < hi_t)
        d = jnp.where(in_range, d, INF)

        for _ in range(K):
            m = jnp.min(d, axis=0, keepdims=True)
            am = jnp.min(jnp.where(d == m, gidx, f32(float(N_S))),
                         axis=0, keepdims=True)
            d = jnp.where(gidx == am, INF, d)
            l1 = (m < rd1) | ((m == rd1) & (am < ri1))
            l2 = (m < rd2) | ((m == rd2) & (am < ri2))
            l3 = (m < rd3) | ((m == rd3) & (am < ri3))
            rd3 = jnp.where(l2, rd2, jnp.where(l3, m, rd3))
            ri3 = jnp.where(l2, ri2, jnp.where(l3, am, ri3))
            rd2 = jnp.where(l1, rd1, jnp.where(l2, m, rd2))
            ri2 = jnp.where(l1, ri1, jnp.where(l2, am, ri2))
            rd1 = jnp.where(l1, m, rd1)
            ri1 = jnp.where(l1, am, ri1)
        return (rd1, ri1, rd2, ri2, rd3, ri3)

    rd1, ri1, rd2, ri2, rd3, ri3 = lax.fori_loop(
        lo_blk, hi_blk, block_body,
        (rd[0], ri[0], rd[1], ri[1], rd[2], ri[2]))

    d3 = jnp.concatenate([rd1, rd2, rd3], axis=0)
    i3 = jnp.concatenate([ri1, ri2, ri3], axis=0)
    w = 1.0 / jnp.maximum(d3, 1e-16)
    w = w / jnp.sum(w, axis=0, keepdims=True)
    idx_ref[...] = jnp.transpose(i3).astype(jnp.int32)
    wT = jnp.transpose(w)
    w_ref[...] = jnp.concatenate(
        [jnp.broadcast_to(wT[:, k:k + 1], (TGT_TILE, 16)) for k in range(K)],
        axis=1)


def _knn(pos_tT, bt, ps, bs):
    n = pos_tT.shape[1]
    grid = n // TGT_TILE
    return pl.pallas_call(
        _knn_body,
        grid=(grid,),
        in_specs=[
            pl.BlockSpec((3, TGT_TILE), lambda i: (0, i)),
            pl.BlockSpec((1, TGT_TILE), lambda i: (0, i)),
            pl.BlockSpec((N_S, 3), lambda i: (0, 0)),
            pl.BlockSpec((1, N_S), lambda i: (0, 0)),
        ],
        out_specs=[
            pl.BlockSpec((TGT_TILE, K), lambda i: (i, 0)),
            pl.BlockSpec((TGT_TILE, 16 * K), lambda i: (i, 0)),
        ],
        out_shape=[
            jax.ShapeDtypeStruct((n, K), jnp.int32),
            jax.ShapeDtypeStruct((n, 16 * K), jnp.float32),
        ],
    )(pos_tT, bt, ps, bs)


_NC = 2
_NS = 16
_NW = _NC * _NS
_TC = 16
_ROWS = _TC * K
_NV = D_S // 16


def _make_sc_body(tpw, nchunk, npair):
  def _sc_gather_body(xs_hbm, idx_hbm, w_hbm, out_hbm,
                      idx_v0, w_v0, rows_v0, out_v0,
                      idx_v1, w_v1, rows_v1, out_v1,
                      si0, sw0, sg0, so0, si1, sw1, sg1, so1):
    wid = lax.axis_index("s") * _NC + lax.axis_index("c")
    base_t = wid * tpw
    bufs = ((idx_v0, w_v0, rows_v0, out_v0, si0, sw0, sg0, so0),
            (idx_v1, w_v1, rows_v1, out_v1, si1, sw1, sg1, so1))

    def meta_copies(c, b):
        idx_v, w_v = bufs[b][0], bufs[b][1]
        t0 = base_t + c * _TC
        ci = pltpu.make_async_copy(idx_hbm.at[pl.ds(t0 * K, _ROWS)], idx_v,
                                   bufs[b][4])
        cw = pltpu.make_async_copy(w_hbm.at[pl.ds(t0, _TC)], w_v, bufs[b][5])
        return ci, cw

    def gather_copy(b):
        return pltpu.make_async_copy(xs_hbm.at[bufs[b][0]], bufs[b][2],
                                     bufs[b][6])

    def out_copy(c, b):
        return pltpu.make_async_copy(bufs[b][3],
                                     out_hbm.at[pl.ds(base_t + c * _TC, _TC)],
                                     bufs[b][7])

    def compute(c, b, p):
        w_v, rows_v, out_v = bufs[b][1], bufs[b][2], bufs[b][3]

        @pl.when(p > 0)
        def _():
            out_copy(c, b).wait()

        def tgt_body(t, carry):
            w0 = w_v[t, pl.ds(0, 16)]
            w1 = w_v[t, pl.ds(16, 16)]
            w2 = w_v[t, pl.ds(32, 16)]
            for v in range(_NV):
                sl = pl.ds(16 * v, 16)
                out_v[t, sl] = (w0 * rows_v[3 * t, sl]
                                + w1 * rows_v[3 * t + 1, sl]
                                + w2 * rows_v[3 * t + 2, sl])
            return carry

        lax.fori_loop(0, _TC, tgt_body, 0)
        out_copy(c, b).start()

    ci, cw = meta_copies(0, 0)
    ci.start(); cw.start(); ci.wait(); cw.wait()
    gather_copy(0).start()
    ci, cw = meta_copies(1, 1)
    ci.start(); cw.start()

    def pair_body(p, carry):
        c0 = 2 * p
        ci, cw = meta_copies(c0 + 1, 1)
        ci.wait(); cw.wait()
        gather_copy(1).start()
        gather_copy(0).wait()
        compute(c0, 0, p)

        @pl.when(p < npair - 1)
        def _():
            ci, cw = meta_copies(c0 + 2, 0)
            ci.start(); cw.start()

        gather_copy(1).wait()
        compute(c0 + 1, 1, p)

        @pl.when(p < npair - 1)
        def _():
            ci, cw = meta_copies(c0 + 3, 1)
            ci.start(); cw.start()

        @pl.when(p < npair - 1)
        def _():
            ci, cw = meta_copies(c0 + 2, 0)
            ci.wait(); cw.wait()
            gather_copy(0).start()
        return carry

    lax.fori_loop(0, npair, pair_body, 0)
    out_copy(nchunk - 2, 0).wait()
    out_copy(nchunk - 1, 1).wait()

  return _sc_gather_body


def _sc_gather(x_source, idx_flat, w_rep):
    n = w_rep.shape[0]
    tpw = n // _NW
    nchunk = tpw // _TC
    mesh = plsc.VectorSubcoreMesh(core_axis_name="c", subcore_axis_name="s")
    buf = [
        pltpu.VMEM((_ROWS,), jnp.int32),
        pltpu.VMEM((_TC, 16 * K), jnp.float32),
        pltpu.VMEM((_ROWS, D_S), jnp.float32),
        pltpu.VMEM((_TC, D_S), jnp.float32),
    ]
    sems = [pltpu.SemaphoreType.DMA] * 4
    kern = functools.partial(
        pl.kernel,
        mesh=mesh,
        out_type=jax.ShapeDtypeStruct((n, D_S), jnp.float32),
        scratch_types=buf + buf + sems + sems,
    )(_make_sc_body(tpw, nchunk, nchunk // 2))
    return kern(x_source, idx_flat, w_rep)


def _mlp_body(xt_ref, it_ref, w1a_ref, w1b_ref, b1_ref,
              w2_ref, b2_ref, wsa_ref, wsb_ref, bs_ref, out_ref):
    xt = xt_ref[...]
    it = it_ref[...]
    f32 = jnp.float32
    dot = functools.partial(jnp.dot, preferred_element_type=f32)
    h = jnp.maximum(dot(xt, w1a_ref[...]) + dot(it, w1b_ref[...])
                    + b1_ref[...], 0.0)
    res = (dot(h, w2_ref[...]) + b2_ref[...]
           + dot(xt, wsa_ref[...]) + dot(it, wsb_ref[...]) + bs_ref[...])
    out_ref[...] = jnp.maximum(res, 0.0)


def _mlp(xt, it, W1a, W1b, b1, W2, b2, Wsa, Wsb, bs):
    n = xt.shape[0]
    grid = n // MLP_TILE
    full = lambda r, c: pl.BlockSpec((r, c), lambda i: (0, 0))
    return pl.pallas_call(
        _mlp_body,
        grid=(grid,),
        in_specs=[
            pl.BlockSpec((MLP_TILE, D_T), lambda i: (i, 0)),
            pl.BlockSpec((MLP_TILE, D_S), lambda i: (i, 0)),
            full(D_T, C_HID), full(D_S, C_HID), full(1, C_HID),
            full(C_HID, C_OUT), full(1, C_OUT),
            full(D_T, C_OUT), full(D_S, C_OUT), full(1, C_OUT),
        ],
        out_specs=pl.BlockSpec((MLP_TILE, C_OUT), lambda i: (i, 0)),
        out_shape=jax.ShapeDtypeStruct((n, C_OUT), jnp.float32),
    )(xt, it, W1a, W1b, b1, W2, b2, Wsa, Wsb, bs)


def kernel(x_target, pos_target, batch_target, x_source, pos_source,
           batch_source, W1, b1, W2, b2, Ws, bs):
    bt_row = batch_target.astype(jnp.int32).reshape(1, N_T)
    bs_row = batch_source.astype(jnp.int32).reshape(1, N_S)
    pt_T = pos_target.T

    HALF = N_T // 4
    outs = []
    for h in range(4):
        s = slice(h * HALF, (h + 1) * HALF)
        idx, w_rep = _knn(pt_T[:, s], bt_row[:, s], pos_source, bs_row)
        interp = _sc_gather(x_source, idx.reshape(-1), w_rep)
        outs.append(_mlp(x_target[s], interp,
                         W1[:D_T], W1[D_T:], b1.reshape(1, C_HID),
                         W2, b2.reshape(1, C_OUT),
                         Ws[:D_T], Ws[D_T:], bs.reshape(1, C_OUT)))
    return jnp.concatenate(outs, axis=0)


# speedup vs baseline: 16.3093x; 1.0512x over previous
import functools

import jax
import jax.numpy as jnp
import numpy as np
from jax import lax
from jax.experimental import pallas as pl
from jax.experimental.pallas import tpu as pltpu
from jax.experimental.pallas import tpu_sc as plsc

N_T, N_S, B = 16384, 4096, 4
D_T, D_S = 256, 512
C_HID, C_OUT = 512, 512
K = 3

TGT_TILE = 512
MLP_TILE = 512
BIG = 1e10


SRC_BLK = 512
N_SRC_BLK = N_S // SRC_BLK


def _knn_body(ptT_ref, bt_ref, ps_ref, bs_ref, idx_ref, w_ref):
    ptT = ptT_ref[...]
    bt = bt_ref[...]
    bs = bs_ref[...]
    sq_t = jnp.sum(ptT * ptT, axis=0, keepdims=True)

    f32 = jnp.float32
    lo_t = jnp.zeros((1, TGT_TILE), f32)
    hi_t = jnp.zeros((1, TGT_TILE), f32)
    cum = jnp.int32(0)
    for b in range(B):
        sel_b = bt == b
        lo_t = jnp.where(sel_b, cum.astype(f32), lo_t)
        cum = cum + jnp.sum((bs == b).astype(jnp.int32))
        hi_t = jnp.where(sel_b, cum.astype(f32), hi_t)
    b_first = jnp.min(bt)
    b_last = jnp.max(bt)
    lo_s = jnp.sum((bs < b_first).astype(jnp.int32))
    hi_s = jnp.sum((bs <= b_last).astype(jnp.int32))
    lo_blk = lo_s // SRC_BLK
    hi_blk = (hi_s + SRC_BLK - 1) // SRC_BLK

    INF = float(np.inf)
    rd = [jnp.full((1, TGT_TILE), BIG, f32) for _ in range(K)]
    ri = [jnp.full((1, TGT_TILE), float(k), f32) for k in range(K)]
    row_iota = lax.broadcasted_iota(
        jnp.int32, (SRC_BLK, TGT_TILE), 0).astype(f32)

    def block_body(j, carry):
        rd1, ri1, rd2, ri2, rd3, ri3 = carry
        psj = ps_ref[pl.ds(j * SRC_BLK, SRC_BLK), :]
        cross = lax.dot_general(psj, ptT, (((1,), (0,)), ((), ())),
                                preferred_element_type=f32)
        sq_s = jnp.sum(psj * psj, axis=1, keepdims=True)
        d = jnp.maximum(sq_s + sq_t - 2.0 * cross, 0.0)
        gidx = row_iota + (j * SRC_BLK).astype(f32)
        in_range = (gidx >= lo_t) & (gidx < hi_t)
        d = jnp.where(in_range, d, INF)

        for _ in range(K):
            m = jnp.min(d, axis=0, keepdims=True)
            am = jnp.min(jnp.where(d == m, gidx, f32(float(N_S))),
                         axis=0, keepdims=True)
            d = jnp.where(gidx == am, INF, d)
            l1 = (m < rd1) | ((m == rd1) & (am < ri1))
            l2 = (m < rd2) | ((m == rd2) & (am < ri2))
            l3 = (m < rd3) | ((m == rd3) & (am < ri3))
            rd3 = jnp.where(l2, rd2, jnp.where(l3, m, rd3))
            ri3 = jnp.where(l2, ri2, jnp.where(l3, am, ri3))
            rd2 = jnp.where(l1, rd1, jnp.where(l2, m, rd2))
            ri2 = jnp.where(l1, ri1, jnp.where(l2, am, ri2))
            rd1 = jnp.where(l1, m, rd1)
            ri1 = jnp.where(l1, am, ri1)
        return (rd1, ri1, rd2, ri2, rd3, ri3)

    rd1, ri1, rd2, ri2, rd3, ri3 = lax.fori_loop(
        lo_blk, hi_blk, block_body,
        (rd[0], ri[0], rd[1], ri[1], rd[2], ri[2]))

    d3 = jnp.concatenate([rd1, rd2, rd3], axis=0)
    i3 = jnp.concatenate([ri1, ri2, ri3], axis=0)
    w = 1.0 / jnp.maximum(d3, 1e-16)
    w = w / jnp.sum(w, axis=0, keepdims=True)
    idx_ref[...] = jnp.transpose(i3).astype(jnp.int32)
    wT = jnp.transpose(w)
    w_ref[...] = jnp.concatenate(
        [jnp.broadcast_to(wT[:, k:k + 1], (TGT_TILE, 16)) for k in range(K)],
        axis=1)


def _knn(pos_tT, bt, ps, bs):
    n = pos_tT.shape[1]
    grid = n // TGT_TILE
    return pl.pallas_call(
        _knn_body,
        grid=(grid,),
        in_specs=[
            pl.BlockSpec((3, TGT_TILE), lambda i: (0, i)),
            pl.BlockSpec((1, TGT_TILE), lambda i: (0, i)),
            pl.BlockSpec((N_S, 3), lambda i: (0, 0)),
            pl.BlockSpec((1, N_S), lambda i: (0, 0)),
        ],
        out_specs=[
            pl.BlockSpec((TGT_TILE, K), lambda i: (i, 0)),
            pl.BlockSpec((TGT_TILE, 16 * K), lambda i: (i, 0)),
        ],
        out_shape=[
            jax.ShapeDtypeStruct((n, K), jnp.int32),
            jax.ShapeDtypeStruct((n, 16 * K), jnp.float32),
        ],
    )(pos_tT, bt, ps, bs)


_NC = 2
_NS = 16
_NW = _NC * _NS
_TC = 16
_ROWS = _TC * K
_NV = D_S // 16


def _make_sc_body(tpw, nchunk, npair):
  def _sc_gather_body(xs_hbm, idx_hbm, w_hbm, out_hbm,
                      idx_v0, w_v0, rows_v0, out_v0,
                      idx_v1, w_v1, rows_v1, out_v1,
                      si0, sw0, sg0, so0, si1, sw1, sg1, so1):
    wid = lax.axis_index("s") * _NC + lax.axis_index("c")
    base_t = wid * tpw
    bufs = ((idx_v0, w_v0, rows_v0, out_v0, si0, sw0, sg0, so0),
            (idx_v1, w_v1, rows_v1, out_v1, si1, sw1, sg1, so1))

    def meta_copies(c, b):
        idx_v, w_v = bufs[b][0], bufs[b][1]
        t0 = base_t + c * _TC
        ci = pltpu.make_async_copy(idx_hbm.at[pl.ds(t0 * K, _ROWS)], idx_v,
                                   bufs[b][4])
        cw = pltpu.make_async_copy(w_hbm.at[pl.ds(t0, _TC)], w_v, bufs[b][5])
        return ci, cw

    def gather_copy(b):
        return pltpu.make_async_copy(xs_hbm.at[bufs[b][0]], bufs[b][2],
                                     bufs[b][6])

    def out_copy(c, b):
        return pltpu.make_async_copy(bufs[b][3],
                                     out_hbm.at[pl.ds(base_t + c * _TC, _TC)],
                                     bufs[b][7])

    def compute(c, b, p):
        w_v, rows_v, out_v = bufs[b][1], bufs[b][2], bufs[b][3]

        @pl.when(p > 0)
        def _():
            out_copy(c, b).wait()

        def tgt_body(t, carry):
            w0 = w_v[t, pl.ds(0, 16)]
            w1 = w_v[t, pl.ds(16, 16)]
            w2 = w_v[t, pl.ds(32, 16)]
            for v in range(_NV):
                sl = pl.ds(16 * v, 16)
                out_v[t, sl] = (w0 * rows_v[3 * t, sl]
                                + w1 * rows_v[3 * t + 1, sl]
                                + w2 * rows_v[3 * t + 2, sl])
            return carry

        lax.fori_loop(0, _TC, tgt_body, 0)
        out_copy(c, b).start()

    ci, cw = meta_copies(0, 0)
    ci.start(); cw.start(); ci.wait(); cw.wait()
    gather_copy(0).start()
    ci, cw = meta_copies(1, 1)
    ci.start(); cw.start()

    def pair_body(p, carry):
        c0 = 2 * p
        ci, cw = meta_copies(c0 + 1, 1)
        ci.wait(); cw.wait()
        gather_copy(1).start()
        gather_copy(0).wait()
        compute(c0, 0, p)

        @pl.when(p < npair - 1)
        def _():
            ci, cw = meta_copies(c0 + 2, 0)
            ci.start(); cw.start()

        gather_copy(1).wait()
        compute(c0 + 1, 1, p)

        @pl.when(p < npair - 1)
        def _():
            ci, cw = meta_copies(c0 + 3, 1)
            ci.start(); cw.start()

        @pl.when(p < npair - 1)
        def _():
            ci, cw = meta_copies(c0 + 2, 0)
            ci.wait(); cw.wait()
            gather_copy(0).start()
        return carry

    lax.fori_loop(0, npair, pair_body, 0)
    out_copy(nchunk - 2, 0).wait()
    out_copy(nchunk - 1, 1).wait()

  return _sc_gather_body


def _sc_gather(x_source, idx_flat, w_rep):
    n = w_rep.shape[0]
    tpw = n // _NW
    nchunk = tpw // _TC
    mesh = plsc.VectorSubcoreMesh(core_axis_name="c", subcore_axis_name="s")
    buf = [
        pltpu.VMEM((_ROWS,), jnp.int32),
        pltpu.VMEM((_TC, 16 * K), jnp.float32),
        pltpu.VMEM((_ROWS, D_S), jnp.float32),
        pltpu.VMEM((_TC, D_S), jnp.float32),
    ]
    sems = [pltpu.SemaphoreType.DMA] * 4
    kern = functools.partial(
        pl.kernel,
        mesh=mesh,
        out_type=jax.ShapeDtypeStruct((n, D_S), jnp.float32),
        scratch_types=buf + buf + sems + sems,
    )(_make_sc_body(tpw, nchunk, nchunk // 2))
    return kern(x_source, idx_flat, w_rep)


def _mlp_body(xt_ref, it_ref, w1a_ref, w1b_ref, b1_ref,
              w2_ref, b2_ref, wsa_ref, wsb_ref, bs_ref, out_ref):
    xt = xt_ref[...]
    it = it_ref[...]
    f32 = jnp.float32
    dot = functools.partial(jnp.dot, preferred_element_type=f32)
    h = jnp.maximum(dot(xt, w1a_ref[...]) + dot(it, w1b_ref[...])
                    + b1_ref[...], 0.0)
    res = (dot(h, w2_ref[...]) + b2_ref[...]
           + dot(xt, wsa_ref[...]) + dot(it, wsb_ref[...]) + bs_ref[...])
    out_ref[...] = jnp.maximum(res, 0.0)


def _mlp(xt, it, W1a, W1b, b1, W2, b2, Wsa, Wsb, bs):
    n = xt.shape[0]
    grid = n // MLP_TILE
    full = lambda r, c: pl.BlockSpec((r, c), lambda i: (0, 0))
    return pl.pallas_call(
        _mlp_body,
        grid=(grid,),
        in_specs=[
            pl.BlockSpec((MLP_TILE, D_T), lambda i: (i, 0)),
            pl.BlockSpec((MLP_TILE, D_S), lambda i: (i, 0)),
            full(D_T, C_HID), full(D_S, C_HID), full(1, C_HID),
            full(C_HID, C_OUT), full(1, C_OUT),
            full(D_T, C_OUT), full(D_S, C_OUT), full(1, C_OUT),
        ],
        out_specs=pl.BlockSpec((MLP_TILE, C_OUT), lambda i: (i, 0)),
        out_shape=jax.ShapeDtypeStruct((n, C_OUT), jnp.float32),
    )(xt, it, W1a, W1b, b1, W2, b2, Wsa, Wsb, bs)


def kernel(x_target, pos_target, batch_target, x_source, pos_source,
           batch_source, W1, b1, W2, b2, Ws, bs):
    bt_row = batch_target.astype(jnp.int32).reshape(1, N_T)
    bs_row = batch_source.astype(jnp.int32).reshape(1, N_S)
    pt_T = pos_target.T

    HALF = N_T // 4
    outs = []
    for h in range(4):
        s = slice(h * HALF, (h + 1) * HALF)
        idx, w_rep = _knn(pt_T[:, s], bt_row[:, s], pos_source, bs_row)
        interp = _sc_gather(x_source, idx.reshape(-1), w_rep)
        outs.append(_mlp(x_target[s], interp,
                         W1[:D_T], W1[D_T:], b1.reshape(1, C_HID),
                         W2, b2.reshape(1, C_OUT),
                         Ws[:D_T], Ws[D_T:], bs.reshape(1, C_OUT)))
    return jnp.concatenate(outs, axis=0)


# speedup vs baseline: 16.8705x; 1.0344x over previous
import functools

import jax
import jax.numpy as jnp
import numpy as np
from jax import lax
from jax.experimental import pallas as pl
from jax.experimental.pallas import tpu as pltpu
from jax.experimental.pallas import tpu_sc as plsc

N_T, N_S, B = 16384, 4096, 4
D_T, D_S = 256, 512
C_HID, C_OUT = 512, 512
K = 3

TGT_TILE = 1024
MLP_TILE = 512
BIG = 1e10


SRC_BLK = 512
N_SRC_BLK = N_S // SRC_BLK


def _knn_body(ptT_ref, bt_ref, ps_ref, bs_ref, idx_ref, w_ref):
    ptT = ptT_ref[...]
    bt = bt_ref[...]
    bs = bs_ref[...]
    sq_t = jnp.sum(ptT * ptT, axis=0, keepdims=True)

    f32 = jnp.float32
    lo_t = jnp.zeros((1, TGT_TILE), f32)
    hi_t = jnp.zeros((1, TGT_TILE), f32)
    cum = jnp.int32(0)
    for b in range(B):
        sel_b = bt == b
        lo_t = jnp.where(sel_b, cum.astype(f32), lo_t)
        cum = cum + jnp.sum((bs == b).astype(jnp.int32))
        hi_t = jnp.where(sel_b, cum.astype(f32), hi_t)
    b_first = jnp.min(bt)
    b_last = jnp.max(bt)
    lo_s = jnp.sum((bs < b_first).astype(jnp.int32))
    hi_s = jnp.sum((bs <= b_last).astype(jnp.int32))
    lo_blk = lo_s // SRC_BLK
    hi_blk = (hi_s + SRC_BLK - 1) // SRC_BLK

    INF = float(np.inf)
    rd = [jnp.full((1, TGT_TILE), BIG, f32) for _ in range(K)]
    ri = [jnp.full((1, TGT_TILE), float(k), f32) for k in range(K)]
    row_iota = lax.broadcasted_iota(
        jnp.int32, (SRC_BLK, TGT_TILE), 0).astype(f32)

    def block_body(j, carry):
        rd1, ri1, rd2, ri2, rd3, ri3 = carry
        psj = ps_ref[pl.ds(j * SRC_BLK, SRC_BLK), :]
        cross = lax.dot_general(psj, ptT, (((1,), (0,)), ((), ())),
                                preferred_element_type=f32)
        sq_s = jnp.sum(psj * psj, axis=1, keepdims=True)
        d = jnp.maximum(sq_s + sq_t - 2.0 * cross, 0.0)
        gidx = row_iota + (j * SRC_BLK).astype(f32)
        in_range = (gidx >= lo_t) & (gidx < hi_t)
        d = jnp.where(in_range, d, INF)

        for _ in range(K):
            m = jnp.min(d, axis=0, keepdims=True)
            am = jnp.min(jnp.where(d == m, gidx, f32(float(N_S))),
                         axis=0, keepdims=True)
            d = jnp.where(gidx == am, INF, d)
            l1 = (m < rd1) | ((m == rd1) & (am < ri1))
            l2 = (m < rd2) | ((m == rd2) & (am < ri2))
            l3 = (m < rd3) | ((m == rd3) & (am < ri3))
            rd3 = jnp.where(l2, rd2, jnp.where(l3, m, rd3))
            ri3 = jnp.where(l2, ri2, jnp.where(l3, am, ri3))
            rd2 = jnp.where(l1, rd1, jnp.where(l2, m, rd2))
            ri2 = jnp.where(l1, ri1, jnp.where(l2, am, ri2))
            rd1 = jnp.where(l1, m, rd1)
            ri1 = jnp.where(l1, am, ri1)
        return (rd1, ri1, rd2, ri2, rd3, ri3)

    rd1, ri1, rd2, ri2, rd3, ri3 = lax.fori_loop(
        lo_blk, hi_blk, block_body,
        (rd[0], ri[0], rd[1], ri[1], rd[2], ri[2]))

    d3 = jnp.concatenate([rd1, rd2, rd3], axis=0)
    i3 = jnp.concatenate([ri1, ri2, ri3], axis=0)
    w = 1.0 / jnp.maximum(d3, 1e-16)
    w = w / jnp.sum(w, axis=0, keepdims=True)
    idx_ref[...] = jnp.transpose(i3).astype(jnp.int32)
    wT = jnp.transpose(w)
    w_ref[...] = jnp.concatenate(
        [jnp.broadcast_to(wT[:, k:k + 1], (TGT_TILE, 16)) for k in range(K)],
        axis=1)


def _knn(pos_tT, bt, ps, bs):
    n = pos_tT.shape[1]
    grid = n // TGT_TILE
    return pl.pallas_call(
        _knn_body,
        grid=(grid,),
        in_specs=[
            pl.BlockSpec((3, TGT_TILE), lambda i: (0, i)),
            pl.BlockSpec((1, TGT_TILE), lambda i: (0, i)),
            pl.BlockSpec((N_S, 3), lambda i: (0, 0)),
            pl.BlockSpec((1, N_S), lambda i: (0, 0)),
        ],
        out_specs=[
            pl.BlockSpec((TGT_TILE, K), lambda i: (i, 0)),
            pl.BlockSpec((TGT_TILE, 16 * K), lambda i: (i, 0)),
        ],
        out_shape=[
            jax.ShapeDtypeStruct((n, K), jnp.int32),
            jax.ShapeDtypeStruct((n, 16 * K), jnp.float32),
        ],
    )(pos_tT, bt, ps, bs)


_NC = 2
_NS = 16
_NW = _NC * _NS
_TC = 16
_ROWS = _TC * K
_NV = D_S // 16


def _make_sc_body(tpw, nchunk, npair):
  def _sc_gather_body(xs_hbm, idx_hbm, w_hbm, out_hbm,
                      idx_v0, w_v0, rows_v0, out_v0,
                      idx_v1, w_v1, rows_v1, out_v1,
                      si0, sw0, sg0, so0, si1, sw1, sg1, so1):
    wid = lax.axis_index("s") * _NC + lax.axis_index("c")
    base_t = wid * tpw
    bufs = ((idx_v0, w_v0, rows_v0, out_v0, si0, sw0, sg0, so0),
            (idx_v1, w_v1, rows_v1, out_v1, si1, sw1, sg1, so1))

    def meta_copies(c, b):
        idx_v, w_v = bufs[b][0], bufs[b][1]
        t0 = base_t + c * _TC
        ci = pltpu.make_async_copy(idx_hbm.at[pl.ds(t0 * K, _ROWS)], idx_v,
                                   bufs[b][4])
        cw = pltpu.make_async_copy(w_hbm.at[pl.ds(t0, _TC)], w_v, bufs[b][5])
        return ci, cw

    def gather_copy(b):
        return pltpu.make_async_copy(xs_hbm.at[bufs[b][0]], bufs[b][2],
                                     bufs[b][6])

    def out_copy(c, b):
        return pltpu.make_async_copy(bufs[b][3],
                                     out_hbm.at[pl.ds(base_t + c * _TC, _TC)],
                                     bufs[b][7])

    def compute(c, b, p):
        w_v, rows_v, out_v = bufs[b][1], bufs[b][2], bufs[b][3]

        @pl.when(p > 0)
        def _():
            out_copy(c, b).wait()

        def tgt_body(t, carry):
            w0 = w_v[t, pl.ds(0, 16)]
            w1 = w_v[t, pl.ds(16, 16)]
            w2 = w_v[t, pl.ds(32, 16)]
            for v in range(_NV):
                sl = pl.ds(16 * v, 16)
                out_v[t, sl] = (w0 * rows_v[3 * t, sl]
                                + w1 * rows_v[3 * t + 1, sl]
                                + w2 * rows_v[3 * t + 2, sl])
            return carry

        lax.fori_loop(0, _TC, tgt_body, 0)
        out_copy(c, b).start()

    ci, cw = meta_copies(0, 0)
    ci.start(); cw.start(); ci.wait(); cw.wait()
    gather_copy(0).start()
    ci, cw = meta_copies(1, 1)
    ci.start(); cw.start()

    def pair_body(p, carry):
        c0 = 2 * p
        ci, cw = meta_copies(c0 + 1, 1)
        ci.wait(); cw.wait()
        gather_copy(1).start()
        gather_copy(0).wait()
        compute(c0, 0, p)

        @pl.when(p < npair - 1)
        def _():
            ci, cw = meta_copies(c0 + 2, 0)
            ci.start(); cw.start()

        gather_copy(1).wait()
        compute(c0 + 1, 1, p)

        @pl.when(p < npair - 1)
        def _():
            ci, cw = meta_copies(c0 + 3, 1)
            ci.start(); cw.start()

        @pl.when(p < npair - 1)
        def _():
            ci, cw = meta_copies(c0 + 2, 0)
            ci.wait(); cw.wait()
            gather_copy(0).start()
        return carry

    lax.fori_loop(0, npair, pair_body, 0)
    out_copy(nchunk - 2, 0).wait()
    out_copy(nchunk - 1, 1).wait()

  return _sc_gather_body


def _sc_gather(x_source, idx_flat, w_rep):
    n = w_rep.shape[0]
    tpw = n // _NW
    nchunk = tpw // _TC
    mesh = plsc.VectorSubcoreMesh(core_axis_name="c", subcore_axis_name="s")
    buf = [
        pltpu.VMEM((_ROWS,), jnp.int32),
        pltpu.VMEM((_TC, 16 * K), jnp.float32),
        pltpu.VMEM((_ROWS, D_S), jnp.float32),
        pltpu.VMEM((_TC, D_S), jnp.float32),
    ]
    sems = [pltpu.SemaphoreType.DMA] * 4
    kern = functools.partial(
        pl.kernel,
        mesh=mesh,
        out_type=jax.ShapeDtypeStruct((n, D_S), jnp.float32),
        scratch_types=buf + buf + sems + sems,
    )(_make_sc_body(tpw, nchunk, nchunk // 2))
    return kern(x_source, idx_flat, w_rep)


def _mlp_body(xt_ref, it_ref, w1a_ref, w1b_ref, b1_ref,
              w2_ref, b2_ref, wsa_ref, wsb_ref, bs_ref, out_ref):
    xt = xt_ref[...]
    it = it_ref[...]
    f32 = jnp.float32
    dot = functools.partial(jnp.dot, preferred_element_type=f32)
    h = jnp.maximum(dot(xt, w1a_ref[...]) + dot(it, w1b_ref[...])
                    + b1_ref[...], 0.0)
    res = (dot(h, w2_ref[...]) + b2_ref[...]
           + dot(xt, wsa_ref[...]) + dot(it, wsb_ref[...]) + bs_ref[...])
    out_ref[...] = jnp.maximum(res, 0.0)


def _mlp(xt, it, W1a, W1b, b1, W2, b2, Wsa, Wsb, bs):
    n = xt.shape[0]
    grid = n // MLP_TILE
    full = lambda r, c: pl.BlockSpec((r, c), lambda i: (0, 0))
    return pl.pallas_call(
        _mlp_body,
        grid=(grid,),
        in_specs=[
            pl.BlockSpec((MLP_TILE, D_T), lambda i: (i, 0)),
            pl.BlockSpec((MLP_TILE, D_S), lambda i: (i, 0)),
            full(D_T, C_HID), full(D_S, C_HID), full(1, C_HID),
            full(C_HID, C_OUT), full(1, C_OUT),
            full(D_T, C_OUT), full(D_S, C_OUT), full(1, C_OUT),
        ],
        out_specs=pl.BlockSpec((MLP_TILE, C_OUT), lambda i: (i, 0)),
        out_shape=jax.ShapeDtypeStruct((n, C_OUT), jnp.float32),
    )(xt, it, W1a, W1b, b1, W2, b2, Wsa, Wsb, bs)


def kernel(x_target, pos_target, batch_target, x_source, pos_source,
           batch_source, W1, b1, W2, b2, Ws, bs):
    bt_row = batch_target.astype(jnp.int32).reshape(1, N_T)
    bs_row = batch_source.astype(jnp.int32).reshape(1, N_S)
    pt_T = pos_target.T

    HALF = N_T // 4
    outs = []
    for h in range(4):
        s = slice(h * HALF, (h + 1) * HALF)
        idx, w_rep = _knn(pt_T[:, s], bt_row[:, s], pos_source, bs_row)
        interp = _sc_gather(x_source, idx.reshape(-1), w_rep)
        outs.append(_mlp(x_target[s], interp,
                         W1[:D_T], W1[D_T:], b1.reshape(1, C_HID),
                         W2, b2.reshape(1, C_OUT),
                         Ws[:D_T], Ws[D_T:], bs.reshape(1, C_OUT)))
    return jnp.concatenate(outs, axis=0)
